# baseline (speedup 1.0000x reference)
import jax
import jax.numpy as jnp
from jax import lax
from jax.experimental import pallas as pl
from jax.experimental.pallas import tpu as pltpu
from jax.experimental.pallas import tpu_sc as plsc

N = 10000
D = 128
DH = 64
E = 320000

NC = 2
NS = 16
NW = NC * NS

C = 128
G = 160
G2 = G // 4
NBUF = 2
EPT = G * C
E_PAD = NS * EPT

RPT = 640
N_PAD = NS * RPT

_mesh = plsc.VectorSubcoreMesh(core_axis_name="c", subcore_axis_name="s")


def _make_agg(with_deg):
  def body(*refs):
    if with_deg:
      (feat, srcr, dstr, zrows, zdeg, ones1, part0, part1, degout,
       feat_sh, agg_sh, src_v, dst_v, rows0, rows1, sem0, sem1,
       deg_sh, ones_v, dsem) = refs
    else:
      (feat, srcr, dstr, zrows, part0, part1,
       feat_sh, agg_sh, src_v, dst_v, rows0, rows1, sem0, sem1) = refs

    cid = lax.axis_index("c")
    sid = lax.axis_index("s")
    row0 = sid * RPT

    bufs = (rows0, rows1)
    sems = (sem0, sem1)

    pltpu.sync_copy(feat.at[cid, pl.ds(row0, RPT)],
                    feat_sh.at[pl.ds(row0, RPT)])
    pltpu.sync_copy(zrows, agg_sh.at[pl.ds(row0, RPT)])
    if with_deg:
      @pl.when(cid == 0)
      def _():
        pltpu.sync_copy(zdeg, deg_sh.at[pl.ds(row0, RPT)])
        pltpu.sync_copy(ones1, ones_v)
    plsc.subcore_barrier()

    def start_g(g, buf, sem):
      pltpu.async_copy(feat_sh.at[src_v.at[g]], buf, sem)

    def wait_g(g, buf, sem):
      pltpu.make_async_copy(feat_sh.at[src_v.at[g]], buf, sem).wait()

    def accum(g, buf):
      pltpu.sync_copy(buf, agg_sh.at[dst_v.at[g]], add=True)
      if with_deg:
        @pl.when(cid == 0)
        def _():
          pltpu.async_copy(ones_v, deg_sh.at[dst_v.at[g]], dsem, add=True)

    def run_slab(h):
      pltpu.sync_copy(srcr.at[sid, pl.ds(h * G2, G2)], src_v)
      pltpu.sync_copy(dstr.at[sid, pl.ds(h * G2, G2)], dst_v)

      for k in range(NBUF):
        start_g(k, bufs[k], sems[k])

      @pl.loop(0, G2, step=NBUF)
      def _(g):
        for k in range(NBUF):
          wait_g(g + k, bufs[k], sems[k])
          accum(g + k, bufs[k])

          @pl.when(g + k + NBUF < G2)
          def _():
            start_g(g + k + NBUF, bufs[k], sems[k])

      if with_deg:
        @pl.when(cid == 0)
        def _():
          @pl.loop(0, G2)
          def _(g):
            pltpu.make_async_copy(ones_v, deg_sh.at[dst_v.at[0]],
                                  dsem).wait()

    run_slab(0)
    run_slab(1)
    run_slab(2)
    run_slab(3)

    plsc.subcore_barrier()

    @pl.when(cid == 0)
    def _():
      pltpu.sync_copy(agg_sh.at[pl.ds(row0, RPT)], part0.at[pl.ds(row0, RPT)])
      if with_deg:
        pltpu.sync_copy(deg_sh.at[pl.ds(row0, RPT)],
                        degout.at[pl.ds(row0, RPT)])

    @pl.when(cid == 1)
    def _():
      pltpu.sync_copy(agg_sh.at[pl.ds(row0, RPT)], part1.at[pl.ds(row0, RPT)])

  out_type = [jax.ShapeDtypeStruct((N_PAD, DH), jnp.float32),
              jax.ShapeDtypeStruct((N_PAD, DH), jnp.float32)]
  scratch = [
      pltpu.VMEM_SHARED((N_PAD, DH), jnp.float32),
      pltpu.VMEM_SHARED((N_PAD, DH), jnp.float32),
      pltpu.VMEM((G2, C), jnp.int32),
      pltpu.VMEM((G2, C), jnp.int32),
      pltpu.VMEM((C, DH), jnp.float32),
      pltpu.VMEM((C, DH), jnp.float32),
      pltpu.SemaphoreType.DMA,
      pltpu.SemaphoreType.DMA,
  ]
  if with_deg:
    out_type = out_type + [jax.ShapeDtypeStruct((N_PAD,), jnp.float32)]
    scratch = scratch + [
        pltpu.VMEM_SHARED((N_PAD,), jnp.float32),
        pltpu.VMEM((C,), jnp.float32),
        pltpu.SemaphoreType.DMA,
    ]
  return pl.kernel(body, out_type=out_type, mesh=_mesh,
                   scratch_types=scratch)


_agg_deg = _make_agg(True)
_agg_half = _make_agg(False)


BN = 1000


def _halves_dot(xs, wl, wr):
  xl = xs[0]
  xr = xs[1]
  h = lax.dot_general(xl, wl, (((1,), (1,)), ((), ())),
                      preferred_element_type=jnp.float32)
  return h + lax.dot_general(xr, wr, (((1,), (1,)), ((), ())),
                             preferred_element_type=jnp.float32)


def _layer_body(a0, a1, d, xs, wll, wlr, bl, wrl, wrr, outs):
  deg = jnp.maximum(d[...], 1.0)
  aggs = jnp.stack([a0[...] / deg, a1[...] / deg])
  h = _halves_dot(aggs, wll[...], wlr[...])
  h = h + bl[...] + _halves_dot(xs[...], wrl[...], wrr[...])
  h = jnp.maximum(h, 0.0)
  outs[...] = jnp.stack([h[:, :DH], h[:, DH:]])


def _layer2_body(a0, a1, d, xs, wll, wlr, bl, wrl, wrr, wc, bc, out):
  deg = jnp.maximum(d[...], 1.0)
  aggs = jnp.stack([a0[...] / deg, a1[...] / deg])
  h = _halves_dot(aggs, wll[...], wlr[...])
  h = h + bl[...] + _halves_dot(xs[...], wrl[...], wrr[...])
  h = jnp.maximum(h, 0.0)
  o = lax.dot_general(h, wc[...], (((1,), (0,)), ((), ())),
                      preferred_element_type=jnp.float32)
  out[...] = o + bc[...]


_half_spec = pl.BlockSpec((BN, DH), lambda i: (i, 0))
_deg_spec = pl.BlockSpec((BN, 1), lambda i: (i, 0))
_stk_spec = pl.BlockSpec((2, BN, DH), lambda i: (0, i, 0))
_wh_spec = pl.BlockSpec((D, DH), lambda i: (0, 0))
_b_spec = pl.BlockSpec((1, D), lambda i: (0, 0))

_layer_tc = pl.pallas_call(
    _layer_body,
    grid=(N // BN,),
    in_specs=[_half_spec, _half_spec, _deg_spec, _stk_spec,
              _wh_spec, _wh_spec, _b_spec, _wh_spec, _wh_spec],
    out_specs=_stk_spec,
    out_shape=jax.ShapeDtypeStruct((2, N_PAD, DH), jnp.float32),
)

_layer2_tc = pl.pallas_call(
    _layer2_body,
    grid=(N // BN,),
    in_specs=[_half_spec, _half_spec, _deg_spec, _stk_spec,
              _wh_spec, _wh_spec, _b_spec, _wh_spec, _wh_spec,
              pl.BlockSpec((D, 1), lambda i: (0, 0)),
              pl.BlockSpec((1, 1), lambda i: (0, 0))],
    out_specs=pl.BlockSpec((BN, 1), lambda i: (i, 0)),
    out_shape=jax.ShapeDtypeStruct((N, 1), jnp.float32),
)


def kernel(x, edge_index, W1_l, b1_l, W1_r, W2_l, b2_l, W2_r, Wc, bc):
  ei = edge_index.astype(jnp.int32)

  pad_a = E_PAD - E
  src_a = jnp.concatenate([ei[0], jnp.zeros((pad_a,), jnp.int32)])
  dst_a = jnp.concatenate([ei[1], jnp.full((pad_a,), N, jnp.int32)])
  srcr = src_a.reshape(NS, G, C)
  dstr = dst_a.reshape(NS, G, C)

  zrows = jnp.zeros((RPT, DH), jnp.float32)
  zdeg = jnp.zeros((RPT,), jnp.float32)
  ones1 = jnp.ones((C,), jnp.float32)

  xp = jnp.pad(x, ((0, N_PAD - N), (0, 0)))
  xs = jnp.stack([xp[:, :DH], xp[:, DH:]])
  p0, p1, gdeg = _agg_deg(xs, srcr, dstr, zrows, zdeg, ones1)
  gd = gdeg.reshape(N_PAD, 1)

  w1ll, w1lr = W1_l[:, :DH], W1_l[:, DH:]
  w1rl, w1rr = W1_r[:, :DH], W1_r[:, DH:]
  hs = _layer_tc(p0, p1, gd, xs,
                 w1ll, w1lr, b1_l.reshape(1, D), w1rl, w1rr)

  q0, q1 = _agg_half(hs, srcr, dstr, zrows)

  w2ll, w2lr = W2_l[:, :DH], W2_l[:, DH:]
  w2rl, w2rr = W2_r[:, :DH], W2_r[:, DH:]
  out = _layer2_tc(q0, q1, gd, hs,
                   w2ll, w2lr, b2_l.reshape(1, D), w2rl, w2rr,
                   Wc.reshape(D, 1), bc.reshape(1, 1))
  return jnp.squeeze(out, axis=-1)


# speedup vs baseline: 1.0283x; 1.0283x over previous
import jax
import jax.numpy as jnp
from jax import lax
from jax.experimental import pallas as pl
from jax.experimental.pallas import tpu as pltpu
from jax.experimental.pallas import tpu_sc as plsc

N = 10000
D = 128
DH = 64
E = 320000

NC = 2
NS = 16
NW = NC * NS

C = 128
G = 160
G2 = G // 4
NBUF = 2
EPT = G * C
E_PAD = NS * EPT

RPT = 640
N_PAD = NS * RPT

_mesh = plsc.VectorSubcoreMesh(core_axis_name="c", subcore_axis_name="s")


def _agg_body(feat, srcr, dstr, zrows, part0, part1,
              feat_sh, agg_sh, src_v, dst_v, rows0, rows1,
              sem0, sem1):
  cid = lax.axis_index("c")
  sid = lax.axis_index("s")
  row0 = sid * RPT

  bufs = (rows0, rows1)
  sems = (sem0, sem1)

  pltpu.sync_copy(feat.at[cid, pl.ds(row0, RPT)],
                  feat_sh.at[pl.ds(row0, RPT)])
  pltpu.sync_copy(zrows, agg_sh.at[pl.ds(row0, RPT)])
  plsc.subcore_barrier()

  def start_g(g, buf, sem):
    pltpu.async_copy(feat_sh.at[src_v.at[g]], buf, sem)

  def wait_g(g, buf, sem):
    pltpu.make_async_copy(feat_sh.at[src_v.at[g]], buf, sem).wait()

  def accum(g, buf):
    pltpu.sync_copy(buf, agg_sh.at[dst_v.at[g]], add=True)

  def run_slab(h):
    pltpu.sync_copy(srcr.at[sid, pl.ds(h * G2, G2)], src_v)
    pltpu.sync_copy(dstr.at[sid, pl.ds(h * G2, G2)], dst_v)

    for k in range(NBUF):
      start_g(k, bufs[k], sems[k])

    @pl.loop(0, G2, step=NBUF)
    def _(g):
      for k in range(NBUF):
        wait_g(g + k, bufs[k], sems[k])
        accum(g + k, bufs[k])

        @pl.when(g + k + NBUF < G2)
        def _():
          start_g(g + k + NBUF, bufs[k], sems[k])

  run_slab(0)
  run_slab(1)
  run_slab(2)
  run_slab(3)

  plsc.subcore_barrier()

  @pl.when(cid == 0)
  def _():
    pltpu.sync_copy(agg_sh.at[pl.ds(row0, RPT)], part0.at[pl.ds(row0, RPT)])

  @pl.when(cid == 1)
  def _():
    pltpu.sync_copy(agg_sh.at[pl.ds(row0, RPT)], part1.at[pl.ds(row0, RPT)])


_agg_half = pl.kernel(
    _agg_body,
    out_type=[jax.ShapeDtypeStruct((N_PAD, DH), jnp.float32),
              jax.ShapeDtypeStruct((N_PAD, DH), jnp.float32)],
    mesh=_mesh,
    scratch_types=[
        pltpu.VMEM_SHARED((N_PAD, DH), jnp.float32),
        pltpu.VMEM_SHARED((N_PAD, DH), jnp.float32),
        pltpu.VMEM((G2, C), jnp.int32),
        pltpu.VMEM((G2, C), jnp.int32),
        pltpu.VMEM((C, DH), jnp.float32),
        pltpu.VMEM((C, DH), jnp.float32),
        pltpu.SemaphoreType.DMA,
        pltpu.SemaphoreType.DMA,
    ],
)


CD = 128
GD = 80
ED_PAD = NW * GD * CD


def _deg_body(dstr, zdeg, ones1, deg0, deg1, deg_sh, dst_v, ones_v):
  cid = lax.axis_index("c")
  sid = lax.axis_index("s")
  wid = sid * NC + cid
  row0 = sid * RPT

  pltpu.sync_copy(zdeg, deg_sh.at[pl.ds(row0, RPT)])
  pltpu.sync_copy(ones1, ones_v)
  pltpu.sync_copy(dstr.at[wid], dst_v)

  plsc.subcore_barrier()

  @pl.loop(0, GD)
  def _(g):
    pltpu.sync_copy(ones_v, deg_sh.at[dst_v.at[g]], add=True)

  plsc.subcore_barrier()

  @pl.when(cid == 0)
  def _():
    pltpu.sync_copy(deg_sh.at[pl.ds(row0, RPT)], deg0.at[pl.ds(row0, RPT)])

  @pl.when(cid == 1)
  def _():
    pltpu.sync_copy(deg_sh.at[pl.ds(row0, RPT)], deg1.at[pl.ds(row0, RPT)])


_deg_kernel = pl.kernel(
    _deg_body,
    out_type=[jax.ShapeDtypeStruct((N_PAD,), jnp.float32),
              jax.ShapeDtypeStruct((N_PAD,), jnp.float32)],
    mesh=_mesh,
    scratch_types=[
        pltpu.VMEM_SHARED((N_PAD,), jnp.float32),
        pltpu.VMEM((GD, CD), jnp.int32),
        pltpu.VMEM((CD,), jnp.float32),
    ],
)


BN = 1000


def _halves_dot(xs, wl, wr):
  xl = xs[0]
  xr = xs[1]
  h = lax.dot_general(xl, wl, (((1,), (1,)), ((), ())),
                      preferred_element_type=jnp.float32)
  return h + lax.dot_general(xr, wr, (((1,), (1,)), ((), ())),
                             preferred_element_type=jnp.float32)


def _layer_body(a0, a1, d0, d1, xs, wll, wlr, bl, wrl, wrr, outs):
  deg = jnp.maximum(d0[...] + d1[...], 1.0)
  aggs = jnp.stack([a0[...] / deg, a1[...] / deg])
  h = _halves_dot(aggs, wll[...], wlr[...])
  h = h + bl[...] + _halves_dot(xs[...], wrl[...], wrr[...])
  h = jnp.maximum(h, 0.0)
  outs[...] = jnp.stack([h[:, :DH], h[:, DH:]])


def _layer2_body(a0, a1, d0, d1, xs, wll, wlr, bl, wrl, wrr, wc, bc, out):
  deg = jnp.maximum(d0[...] + d1[...], 1.0)
  aggs = jnp.stack([a0[...] / deg, a1[...] / deg])
  h = _halves_dot(aggs, wll[...], wlr[...])
  h = h + bl[...] + _halves_dot(xs[...], wrl[...], wrr[...])
  h = jnp.maximum(h, 0.0)
  o = lax.dot_general(h, wc[...], (((1,), (0,)), ((), ())),
                      preferred_element_type=jnp.float32)
  out[...] = o + bc[...]


_half_spec = pl.BlockSpec((BN, DH), lambda i: (i, 0))
_deg_spec = pl.BlockSpec((BN, 1), lambda i: (i, 0))
_stk_spec = pl.BlockSpec((2, BN, DH), lambda i: (0, i, 0))
_wh_spec = pl.BlockSpec((D, DH), lambda i: (0, 0))
_b_spec = pl.BlockSpec((1, D), lambda i: (0, 0))

_layer_tc = pl.pallas_call(
    _layer_body,
    grid=(N // BN,),
    in_specs=[_half_spec, _half_spec, _deg_spec, _deg_spec, _stk_spec,
              _wh_spec, _wh_spec, _b_spec, _wh_spec, _wh_spec],
    out_specs=_stk_spec,
    out_shape=jax.ShapeDtypeStruct((2, N_PAD, DH), jnp.float32),
)

_layer2_tc = pl.pallas_call(
    _layer2_body,
    grid=(N // BN,),
    in_specs=[_half_spec, _half_spec, _deg_spec, _deg_spec, _stk_spec,
              _wh_spec, _wh_spec, _b_spec, _wh_spec, _wh_spec,
              pl.BlockSpec((D, 1), lambda i: (0, 0)),
              pl.BlockSpec((1, 1), lambda i: (0, 0))],
    out_specs=pl.BlockSpec((BN, 1), lambda i: (i, 0)),
    out_shape=jax.ShapeDtypeStruct((N, 1), jnp.float32),
)


def kernel(x, edge_index, W1_l, b1_l, W1_r, W2_l, b2_l, W2_r, Wc, bc):
  ei = edge_index.astype(jnp.int32)

  pad_a = E_PAD - E
  src_a = jnp.concatenate([ei[0], jnp.zeros((pad_a,), jnp.int32)])
  dst_a = jnp.concatenate([ei[1], jnp.full((pad_a,), N, jnp.int32)])
  srcr = src_a.reshape(NS, G, C)
  dstr = dst_a.reshape(NS, G, C)

  pad_d = ED_PAD - E
  dst_d = jnp.concatenate([ei[1], jnp.full((pad_d,), N, jnp.int32)])
  dstr_d = dst_d.reshape(NW, GD, CD)

  zrows = jnp.zeros((RPT, DH), jnp.float32)
  zdeg = jnp.zeros((RPT,), jnp.float32)
  ones1 = jnp.ones((CD,), jnp.float32)

  g0, g1 = _deg_kernel(dstr_d, zdeg, ones1)
  g0 = g0.reshape(N_PAD, 1)
  g1 = g1.reshape(N_PAD, 1)

  xp = jnp.pad(x, ((0, N_PAD - N), (0, 0)))
  xs = jnp.stack([xp[:, :DH], xp[:, DH:]])
  p0, p1 = _agg_half(xs, srcr, dstr, zrows)

  w1ll, w1lr = W1_l[:, :DH], W1_l[:, DH:]
  w1rl, w1rr = W1_r[:, :DH], W1_r[:, DH:]
  hs = _layer_tc(p0, p1, g0, g1, xs,
                 w1ll, w1lr, b1_l.reshape(1, D), w1rl, w1rr)

  q0, q1 = _agg_half(hs, srcr, dstr, zrows)

  w2ll, w2lr = W2_l[:, :DH], W2_l[:, DH:]
  w2rl, w2rr = W2_r[:, :DH], W2_r[:, DH:]
  out = _layer2_tc(q0, q1, g0, g1, hs,
                   w2ll, w2lr, b2_l.reshape(1, D), w2rl, w2rr,
                   Wc.reshape(D, 1), bc.reshape(1, 1))
  return jnp.squeeze(out, axis=-1)
